# baseline (speedup 1.0000x reference)
import functools

import jax
import jax.numpy as jnp
from jax import lax
from jax.experimental import pallas as pl
from jax.experimental.pallas import tpu as pltpu
from jax.experimental.pallas import tpu_sc as plsc

B = 16384
D = 128
LANES = 16
NUM_CORES = 2
NUM_SUBCORES = 16
NW = NUM_CORES * NUM_SUBCORES
BPW = B // NW
CHUNK = 64
NCHUNK = BPW // CHUNK
NSUPER = NCHUNK // 2
NGROUP = CHUNK // LANES


def _row_acc(hb, rb, tb, i):
    acc = (hb[i, pl.ds(0, LANES)]
           * rb[i, pl.ds(0, LANES)]
           * tb[i, pl.ds(0, LANES)])
    for k in range(1, D // LANES):
        o = k * LANES
        acc = acc + (hb[i, pl.ds(o, LANES)]
                     * rb[i, pl.ds(o, LANES)]
                     * tb[i, pl.ds(o, LANES)])
    return acc


def _perm(x, idx):
    return jnp.take_along_axis(x, idx, axis=0, mode="promise_in_bounds")


def _compute_chunk(hb, rb, tb, scores, cb, lane):
    def group_body(g, carry):
        gb = g * LANES
        stack = []
        for j in range(LANES):
            v = _row_acc(hb, rb, tb, gb + j)
            lvl = 0
            while stack and stack[-1][0] == lvl:
                _, pv = stack.pop()
                s = 1 << lvl
                pa = pv + _perm(pv, lane ^ s)
                pb = v + _perm(v, lane ^ s)
                v = jnp.where((lane & s) == 0, pa, pb)
                lvl += 1
            stack.append((lvl, v))
        scores[pl.ds(cb + gb, LANES)] = stack[0][1]
        return carry

    lax.fori_loop(0, NGROUP, group_body, 0)


def _sc_kernel(head_hbm, rel_hbm, tail_hbm, ent_hbm, relemb_hbm, out_hbm,
               hidx, ridx, tidx, scores, bufs, sems):
    wid = lax.axis_index("s") * NUM_CORES + lax.axis_index("c")
    base = wid * BPW

    pltpu.sync_copy(head_hbm.at[pl.ds(base, BPW)], hidx)
    pltpu.sync_copy(rel_hbm.at[pl.ds(base, BPW)], ridx)
    pltpu.sync_copy(tail_hbm.at[pl.ds(base, BPW)], tidx)

    lane = lax.iota(jnp.int32, LANES)

    def copies(ci, bset, sset):
        cb = ci * CHUNK
        hb, rb, tb = bset
        sh, sr, st = sset
        return (
            pltpu.make_async_copy(ent_hbm.at[hidx.at[pl.ds(cb, CHUNK)]], hb, sh),
            pltpu.make_async_copy(relemb_hbm.at[ridx.at[pl.ds(cb, CHUNK)]], rb, sr),
            pltpu.make_async_copy(ent_hbm.at[tidx.at[pl.ds(cb, CHUNK)]], tb, st),
        )

    def start(ci, bset, sset):
        for c in copies(ci, bset, sset):
            c.start()

    def wait(ci, bset, sset):
        for c in copies(ci, bset, sset):
            c.wait()

    start(0, bufs[0], sems[0])

    def superstep(t, carry):
        c0 = 2 * t
        start(c0 + 1, bufs[1], sems[1])
        wait(c0, bufs[0], sems[0])
        _compute_chunk(bufs[0][0], bufs[0][1], bufs[0][2],
                       scores, c0 * CHUNK, lane)

        @pl.when(t + 1 < NSUPER)
        def _():
            start(c0 + 2, bufs[0], sems[0])

        wait(c0 + 1, bufs[1], sems[1])
        _compute_chunk(bufs[1][0], bufs[1][1], bufs[1][2],
                       scores, (c0 + 1) * CHUNK, lane)
        return carry

    lax.fori_loop(0, NSUPER, superstep, 0)
    pltpu.sync_copy(scores, out_hbm.at[pl.ds(base, BPW)])


@functools.partial(
    pl.kernel,
    mesh=plsc.VectorSubcoreMesh(core_axis_name="c", subcore_axis_name="s"),
    out_type=jax.ShapeDtypeStruct((B,), jnp.float32),
    scratch_types=[
        pltpu.VMEM((BPW,), jnp.int32),
        pltpu.VMEM((BPW,), jnp.int32),
        pltpu.VMEM((BPW,), jnp.int32),
        pltpu.VMEM((BPW,), jnp.float32),
    ] + [pltpu.VMEM((CHUNK, D), jnp.float32) for _ in range(6)]
      + [pltpu.SemaphoreType.DMA for _ in range(6)],
)
def _distmult_sc(head_hbm, rel_hbm, tail_hbm, ent_hbm, relemb_hbm, out_hbm,
                 hidx, ridx, tidx, scores,
                 hb0, rb0, tb0, hb1, rb1, tb1,
                 sh0, sr0, st0, sh1, sr1, st1):
    _sc_kernel(head_hbm, rel_hbm, tail_hbm, ent_hbm, relemb_hbm, out_hbm,
               hidx, ridx, tidx, scores,
               ((hb0, rb0, tb0), (hb1, rb1, tb1)),
               ((sh0, sr0, st0), (sh1, sr1, st1)))


def kernel(head, relation, tail, entity_emb, relation_emb):
    head = head.astype(jnp.int32)
    relation = relation.astype(jnp.int32)
    tail = tail.astype(jnp.int32)
    return _distmult_sc(head, relation, tail, entity_emb, relation_emb)


# speedup vs baseline: 1.7246x; 1.7246x over previous
import functools

import jax
import jax.numpy as jnp
from jax import lax
from jax.experimental import pallas as pl
from jax.experimental.pallas import tpu as pltpu
from jax.experimental.pallas import tpu_sc as plsc

B = 16384
D = 128
LANES = 16
NUM_CORES = 2
NUM_SUBCORES = 16
NW = NUM_CORES * NUM_SUBCORES
BPW = B // NW
CHUNK = 64
NCHUNK = BPW // CHUNK
NSUPER = NCHUNK // 2
NGROUP = CHUNK // LANES


def _row_acc(hb, rb, tb, i):
    acc = (hb[i, pl.ds(0, LANES)]
           * rb[i, pl.ds(0, LANES)]
           * tb[i, pl.ds(0, LANES)])
    for k in range(1, D // LANES):
        o = k * LANES
        acc = acc + (hb[i, pl.ds(o, LANES)]
                     * rb[i, pl.ds(o, LANES)]
                     * tb[i, pl.ds(o, LANES)])
    return acc


def _perm(x, idx):
    return jnp.take_along_axis(x, idx, axis=0, mode="promise_in_bounds")


def _compute_chunk(hb, rb, tb, scores, cb, lane):
    def group_body(g, carry):
        gb = g * LANES

        def row_body(j, vec):
            acc = _row_acc(hb, rb, tb, gb + j)
            for s in (8, 4, 2, 1):
                acc = acc + _perm(acc, lane ^ s)
            return jnp.where(lane == j, acc, vec)

        vec = lax.fori_loop(0, LANES, row_body,
                            jnp.zeros((LANES,), jnp.float32), unroll=4)
        scores[pl.ds(cb + gb, LANES)] = vec
        return carry

    lax.fori_loop(0, NGROUP, group_body, 0)


def _sc_kernel(head_hbm, rel_hbm, tail_hbm, ent_hbm, relemb_hbm, out_hbm,
               hidx, ridx, tidx, scores, bufs, sems):
    wid = lax.axis_index("s") * NUM_CORES + lax.axis_index("c")
    base = wid * BPW

    pltpu.sync_copy(head_hbm.at[pl.ds(base, BPW)], hidx)
    pltpu.sync_copy(rel_hbm.at[pl.ds(base, BPW)], ridx)
    pltpu.sync_copy(tail_hbm.at[pl.ds(base, BPW)], tidx)

    lane = lax.iota(jnp.int32, LANES)

    def copies(ci, bset, sset):
        cb = ci * CHUNK
        hb, rb, tb = bset
        sh, sr, st = sset
        return (
            pltpu.make_async_copy(ent_hbm.at[hidx.at[pl.ds(cb, CHUNK)]], hb, sh),
            pltpu.make_async_copy(relemb_hbm.at[ridx.at[pl.ds(cb, CHUNK)]], rb, sr),
            pltpu.make_async_copy(ent_hbm.at[tidx.at[pl.ds(cb, CHUNK)]], tb, st),
        )

    def start(ci, bset, sset):
        for c in copies(ci, bset, sset):
            c.start()

    def wait(ci, bset, sset):
        for c in copies(ci, bset, sset):
            c.wait()

    start(0, bufs[0], sems[0])

    def superstep(t, carry):
        c0 = 2 * t
        start(c0 + 1, bufs[1], sems[1])
        wait(c0, bufs[0], sems[0])
        _compute_chunk(bufs[0][0], bufs[0][1], bufs[0][2],
                       scores, c0 * CHUNK, lane)

        @pl.when(t + 1 < NSUPER)
        def _():
            start(c0 + 2, bufs[0], sems[0])

        wait(c0 + 1, bufs[1], sems[1])
        _compute_chunk(bufs[1][0], bufs[1][1], bufs[1][2],
                       scores, (c0 + 1) * CHUNK, lane)
        return carry

    lax.fori_loop(0, NSUPER, superstep, 0)
    pltpu.sync_copy(scores, out_hbm.at[pl.ds(base, BPW)])


@functools.partial(
    pl.kernel,
    mesh=plsc.VectorSubcoreMesh(core_axis_name="c", subcore_axis_name="s"),
    out_type=jax.ShapeDtypeStruct((B,), jnp.float32),
    scratch_types=[
        pltpu.VMEM((BPW,), jnp.int32),
        pltpu.VMEM((BPW,), jnp.int32),
        pltpu.VMEM((BPW,), jnp.int32),
        pltpu.VMEM((BPW,), jnp.float32),
    ] + [pltpu.VMEM((CHUNK, D), jnp.float32) for _ in range(6)]
      + [pltpu.SemaphoreType.DMA for _ in range(6)],
)
def _distmult_sc(head_hbm, rel_hbm, tail_hbm, ent_hbm, relemb_hbm, out_hbm,
                 hidx, ridx, tidx, scores,
                 hb0, rb0, tb0, hb1, rb1, tb1,
                 sh0, sr0, st0, sh1, sr1, st1):
    _sc_kernel(head_hbm, rel_hbm, tail_hbm, ent_hbm, relemb_hbm, out_hbm,
               hidx, ridx, tidx, scores,
               ((hb0, rb0, tb0), (hb1, rb1, tb1)),
               ((sh0, sr0, st0), (sh1, sr1, st1)))


def kernel(head, relation, tail, entity_emb, relation_emb):
    head = head.astype(jnp.int32)
    relation = relation.astype(jnp.int32)
    tail = tail.astype(jnp.int32)
    return _distmult_sc(head, relation, tail, entity_emb, relation_emb)


# speedup vs baseline: 1.7281x; 1.0020x over previous
import functools

import jax
import jax.numpy as jnp
from jax import lax
from jax.experimental import pallas as pl
from jax.experimental.pallas import tpu as pltpu
from jax.experimental.pallas import tpu_sc as plsc

B = 16384
D = 128
LANES = 16
NUM_CORES = 2
NUM_SUBCORES = 16
NW = NUM_CORES * NUM_SUBCORES
BPW = B // NW
CHUNK = 64
NCHUNK = BPW // CHUNK
NSUPER = NCHUNK // 2
NGROUP = CHUNK // LANES


def _row_acc(hb, rb, tb, i):
    acc = (hb[i, pl.ds(0, LANES)]
           * rb[i, pl.ds(0, LANES)]
           * tb[i, pl.ds(0, LANES)])
    for k in range(1, D // LANES):
        o = k * LANES
        acc = acc + (hb[i, pl.ds(o, LANES)]
                     * rb[i, pl.ds(o, LANES)]
                     * tb[i, pl.ds(o, LANES)])
    return acc


def _perm(x, idx):
    return jnp.take_along_axis(x, idx, axis=0, mode="promise_in_bounds")


def _compute_chunk(hb, rb, tb, partials, scores, cb, lane):
    def row_body(i, carry):
        partials[i, pl.ds(0, LANES)] = _row_acc(hb, rb, tb, i)
        return carry

    lax.fori_loop(0, CHUNK, row_body, 0, unroll=4)

    def merge(a, b, s):
        mask = (lane & s) == 0
        return jnp.where(mask, a, b) + _perm(jnp.where(mask, b, a),
                                             lane ^ s)

    def group_body(g, carry):
        gb = g * LANES
        stack = []
        for j in range(LANES):
            v = partials[gb + j, pl.ds(0, LANES)]
            lvl = 0
            while stack and stack[-1][0] == lvl:
                _, pv = stack.pop()
                v = merge(pv, v, 1 << lvl)
                lvl += 1
            stack.append((lvl, v))
        scores[pl.ds(cb + gb, LANES)] = stack[0][1]
        return carry

    lax.fori_loop(0, NGROUP, group_body, 0)


def _sc_kernel(head_hbm, rel_hbm, tail_hbm, ent_hbm, relemb_hbm, out_hbm,
               hidx, ridx, tidx, scores, partials, bufs, sems):
    wid = lax.axis_index("s") * NUM_CORES + lax.axis_index("c")
    base = wid * BPW

    idx_cps = (
        pltpu.make_async_copy(head_hbm.at[pl.ds(base, BPW)], hidx, sems[0][0]),
        pltpu.make_async_copy(rel_hbm.at[pl.ds(base, BPW)], ridx, sems[0][1]),
        pltpu.make_async_copy(tail_hbm.at[pl.ds(base, BPW)], tidx, sems[0][2]),
    )
    for c in idx_cps:
        c.start()
    for c in idx_cps:
        c.wait()

    lane = lax.iota(jnp.int32, LANES)

    def copies(ci, bset, sset):
        cb = ci * CHUNK
        hb, rb, tb = bset
        sh, sr, st = sset
        return (
            pltpu.make_async_copy(ent_hbm.at[hidx.at[pl.ds(cb, CHUNK)]], hb, sh),
            pltpu.make_async_copy(relemb_hbm.at[ridx.at[pl.ds(cb, CHUNK)]], rb, sr),
            pltpu.make_async_copy(ent_hbm.at[tidx.at[pl.ds(cb, CHUNK)]], tb, st),
        )

    def start(ci, bset, sset):
        for c in copies(ci, bset, sset):
            c.start()

    def wait(ci, bset, sset):
        for c in copies(ci, bset, sset):
            c.wait()

    start(0, bufs[0], sems[0])

    def superstep(t, carry):
        c0 = 2 * t
        start(c0 + 1, bufs[1], sems[1])
        wait(c0, bufs[0], sems[0])
        _compute_chunk(bufs[0][0], bufs[0][1], bufs[0][2],
                       partials, scores, c0 * CHUNK, lane)

        @pl.when(t + 1 < NSUPER)
        def _():
            start(c0 + 2, bufs[0], sems[0])

        wait(c0 + 1, bufs[1], sems[1])
        _compute_chunk(bufs[1][0], bufs[1][1], bufs[1][2],
                       partials, scores, (c0 + 1) * CHUNK, lane)
        return carry

    lax.fori_loop(0, NSUPER, superstep, 0)
    pltpu.sync_copy(scores, out_hbm.at[pl.ds(base, BPW)])


@functools.partial(
    pl.kernel,
    mesh=plsc.VectorSubcoreMesh(core_axis_name="c", subcore_axis_name="s"),
    out_type=jax.ShapeDtypeStruct((B,), jnp.float32),
    scratch_types=[
        pltpu.VMEM((BPW,), jnp.int32),
        pltpu.VMEM((BPW,), jnp.int32),
        pltpu.VMEM((BPW,), jnp.int32),
        pltpu.VMEM((BPW,), jnp.float32),
        pltpu.VMEM((CHUNK, LANES), jnp.float32),
    ] + [pltpu.VMEM((CHUNK, D), jnp.float32) for _ in range(6)]
      + [pltpu.SemaphoreType.DMA for _ in range(6)],
)
def _distmult_sc(head_hbm, rel_hbm, tail_hbm, ent_hbm, relemb_hbm, out_hbm,
                 hidx, ridx, tidx, scores, partials,
                 hb0, rb0, tb0, hb1, rb1, tb1,
                 sh0, sr0, st0, sh1, sr1, st1):
    _sc_kernel(head_hbm, rel_hbm, tail_hbm, ent_hbm, relemb_hbm, out_hbm,
               hidx, ridx, tidx, scores, partials,
               ((hb0, rb0, tb0), (hb1, rb1, tb1)),
               ((sh0, sr0, st0), (sh1, sr1, st1)))


def kernel(head, relation, tail, entity_emb, relation_emb):
    head = head.astype(jnp.int32)
    relation = relation.astype(jnp.int32)
    tail = tail.astype(jnp.int32)
    return _distmult_sc(head, relation, tail, entity_emb, relation_emb)
